# Initial kernel scaffold; baseline (speedup 1.0000x reference)
import jax
import jax.numpy as jnp
from jax.experimental import pallas as pl


def kernel(x, edge_index, edge_weight, node_weight, graph_ids, W_conv, gn_gamma, gn_beta, gn_alpha, W_lin, W_cls):
    raise NotImplementedError("write your pallas kernel here")


# speedup vs baseline: 6.5021x; 6.5021x over previous
import functools

import jax
import jax.numpy as jnp
from jax import lax
from jax.experimental import pallas as pl
from jax.experimental.pallas import tpu as pltpu
from jax.experimental.pallas import tpu_sc as plsc

N = 10000
E = 320000
F = 128
H = 256
G = 64
OUT = 10

NC = 2
NS = 16
NW = NC * NS
CE = 128
NCHUNK = E // CE
CH_BASE = NCHUNK // NW
CH_REM = NCHUNK % NW
NP = 10240
ZB = NP // NS

def _deg_body(src_hbm, dst_hbm, out_hbm, sbuf, dbuf, ones, zbuf,
              acc_src, acc_dst):
    cid = lax.axis_index("c")
    sid = lax.axis_index("s")
    wid = sid * NC + cid
    for i in range(CE // 16):
        ones[pl.ds(16 * i, 16)] = jnp.ones((16,), jnp.float32)
    for i in range(ZB // 16):
        zbuf[pl.ds(16 * i, 16)] = jnp.zeros((16,), jnp.float32)
    pltpu.sync_copy(zbuf, acc_src.at[pl.ds(sid * ZB, ZB)])
    pltpu.sync_copy(zbuf, acc_dst.at[pl.ds(sid * ZB, ZB)])
    plsc.subcore_barrier()

    nch = CH_BASE + jnp.where(wid < CH_REM, 1, 0)

    def body(i, carry):
        base = (wid + i * NW) * CE
        pltpu.sync_copy(src_hbm.at[pl.ds(base, CE)], sbuf)
        pltpu.sync_copy(dst_hbm.at[pl.ds(base, CE)], dbuf)
        pltpu.sync_copy(ones, acc_src.at[sbuf], add=True)
        pltpu.sync_copy(ones, acc_dst.at[dbuf], add=True)
        return carry

    lax.fori_loop(0, nch, body, 0)
    plsc.subcore_barrier()
    pltpu.sync_copy(acc_src.at[pl.ds(sid * ZB, ZB)],
                    out_hbm.at[cid, 0, pl.ds(sid * ZB, ZB)])
    pltpu.sync_copy(acc_dst.at[pl.ds(sid * ZB, ZB)],
                    out_hbm.at[cid, 1, pl.ds(sid * ZB, ZB)])


def _spmm_body(src_hbm, dst_hbm, ew_hbm, xs_hbm, out_hbm,
               sbuf, dbuf, ewbuf, rows, zbuf, acc, sem):
    cid = lax.axis_index("c")
    sid = lax.axis_index("s")
    wid = sid * NC + cid

    def zfill(i, carry):
        zbuf[i, pl.ds(0, 16)] = jnp.zeros((16,), jnp.float32)
        for k in range(1, F // 16):
            zbuf[i, pl.ds(16 * k, 16)] = jnp.zeros((16,), jnp.float32)
        return carry

    lax.fori_loop(0, 64, zfill, 0)
    for j in range(ZB // 64):
        pltpu.sync_copy(zbuf, acc.at[pl.ds(sid * ZB + j * 64, 64), :])
    plsc.subcore_barrier()

    nch = CH_BASE + jnp.where(wid < CH_REM, 1, 0)

    def body(i, carry):
        base = (wid + i * NW) * CE
        pltpu.sync_copy(src_hbm.at[pl.ds(base, CE)], sbuf)
        pltpu.sync_copy(dst_hbm.at[pl.ds(base, CE)], dbuf)
        pltpu.sync_copy(ew_hbm.at[pl.ds(base, CE)], ewbuf)
        pltpu.async_copy(xs_hbm.at[sbuf], rows, sem).wait()

        def escale(e16, c2):
            wv = ewbuf[pl.ds(e16 * 16, 16)]
            for j in range(16):
                wj = wv.at[jnp.full((16,), j, jnp.int32)].get(
                    mode='promise_in_bounds')
                for k in range(F // 16):
                    sl = pl.ds(16 * k, 16)
                    rows[e16 * 16 + j, sl] = rows[e16 * 16 + j, sl] * wj
            return c2

        lax.fori_loop(0, CE // 16, escale, 0)
        pltpu.sync_copy(rows, acc.at[dbuf], add=True)
        return carry

    lax.fori_loop(0, nch, body, 0)
    plsc.subcore_barrier()
    for j in range(ZB // 64):
        pltpu.sync_copy(acc.at[pl.ds(sid * ZB + j * 64, 64), :],
                        out_hbm.at[cid, pl.ds(sid * ZB + j * 64, 64), :])


def _xs_body(x_ref, degp_ref, xs_ref):
    dout = degp_ref[0, :N, :] + degp_ref[2, :N, :]
    scale = lax.rsqrt(jnp.maximum(dout, 1.0))
    xs_ref[...] = x_ref[...] * scale


def _leaky(z):
    return jnp.where(z >= 0, z, 0.01 * z)


_HI = lax.Precision.HIGHEST


BN = 2000


def _mom_body(p_ref, degp_ref, nw_ref, gid_ref, wc_ref,
              s1_ref, s2_ref, t1_ref, c_ref):
    f32 = jnp.float32
    agg = p_ref[0] + p_ref[1]
    din = degp_ref[1] + degp_ref[3]
    sin = lax.rsqrt(jnp.maximum(din, 1.0))
    h = jnp.dot(agg, wc_ref[...], preferred_element_type=f32,
                precision=_HI) * sin
    h = _leaky(h)

    ids = gid_ref[...]
    iota_g = lax.broadcasted_iota(jnp.int32, (BN, G), 1)
    m = (ids == iota_g).astype(f32)

    seg = lambda v: lax.dot_general(
        m, v, (((0,), (0,)), ((), ())), preferred_element_type=f32,
        precision=_HI)

    nw = nw_ref[...]
    onw = jnp.concatenate([jnp.ones((BN, 1), f32), nw], axis=1)

    @pl.when(pl.program_id(0) == 0)
    def _init():
        s1_ref[...] = jnp.zeros_like(s1_ref)
        s2_ref[...] = jnp.zeros_like(s2_ref)
        t1_ref[...] = jnp.zeros_like(t1_ref)
        c_ref[...] = jnp.zeros_like(c_ref)

    s1_ref[...] += seg(h)
    s2_ref[...] += seg(h * h)
    t1_ref[...] += seg(nw * h)
    c_ref[...] += seg(onw)


def _fin_body(s1_ref, s2_ref, t1_ref, c_ref, gg_ref, gb_ref, ga_ref,
              wl_ref, wcls_ref, out_ref):
    f32 = jnp.float32
    cnt = jnp.maximum(c_ref[:, 0:1], 1.0)
    wg = c_ref[:, 1:2]
    inv = 1.0 / cnt
    a = ga_ref[...]
    mean = s1_ref[...] * inv
    var = s2_ref[...] * inv - (2.0 * a - a * a) * mean * mean
    hscale = gg_ref[...] * lax.rsqrt(var + 1e-5)
    r = (hscale * (t1_ref[...] - a * mean * wg) + gb_ref[...] * wg) * inv
    r2 = _leaky(jnp.dot(r, wl_ref[...], preferred_element_type=f32,
                        precision=_HI))
    mu = jnp.mean(r2, axis=1, keepdims=True)
    v = jnp.mean((r2 - mu) ** 2, axis=1, keepdims=True)
    rn = (r2 - mu) * lax.rsqrt(v + 1e-5)
    out_ref[...] = jnp.dot(rn, wcls_ref[...], preferred_element_type=f32,
                           precision=_HI)


@functools.lru_cache(maxsize=None)
def _build_sc_kernels():
    mesh = plsc.VectorSubcoreMesh(
        core_axis_name="c", subcore_axis_name="s",
        num_cores=NC, num_subcores=NS)
    deg = pl.kernel(
        _deg_body,
        out_type=jax.ShapeDtypeStruct((NC, 2, NP), jnp.float32),
        mesh=mesh,
        scratch_types=[
            pltpu.VMEM((CE,), jnp.int32),
            pltpu.VMEM((CE,), jnp.int32),
            pltpu.VMEM((CE,), jnp.float32),
            pltpu.VMEM((ZB,), jnp.float32),
            pltpu.VMEM_SHARED((NP,), jnp.float32),
            pltpu.VMEM_SHARED((NP,), jnp.float32),
        ],
    )
    spmm = pl.kernel(
        _spmm_body,
        out_type=jax.ShapeDtypeStruct((NC, NP, F), jnp.float32),
        mesh=mesh,
        scratch_types=[
            pltpu.VMEM((CE,), jnp.int32),
            pltpu.VMEM((CE,), jnp.int32),
            pltpu.VMEM((CE,), jnp.float32),
            pltpu.VMEM((CE, F), jnp.float32),
            pltpu.VMEM((64, F), jnp.float32),
            pltpu.VMEM_SHARED((NP, F), jnp.float32),
            pltpu.SemaphoreType.DMA,
        ],
    )
    return deg, spmm


def kernel(x, edge_index, edge_weight, node_weight, graph_ids, W_conv,
           gn_gamma, gn_beta, gn_alpha, W_lin, W_cls):
    src = edge_index[0]
    dst = edge_index[1]
    _deg_kernel, _spmm_kernel = _build_sc_kernels()

    degp = _deg_kernel(src, dst)
    degp4 = degp.reshape(4, NP, 1)

    xs = pl.pallas_call(
        _xs_body,
        out_shape=jax.ShapeDtypeStruct((N, F), jnp.float32),
    )(x, degp4)

    p = _spmm_kernel(src, dst, edge_weight, xs)

    s1, s2, t1, c = pl.pallas_call(
        _mom_body,
        grid=(N // BN,),
        in_specs=[
            pl.BlockSpec((2, BN, F), lambda i: (0, i, 0)),
            pl.BlockSpec((4, BN, 1), lambda i: (0, i, 0)),
            pl.BlockSpec((BN, 1), lambda i: (i, 0)),
            pl.BlockSpec((BN, 1), lambda i: (i, 0)),
            pl.BlockSpec((F, H), lambda i: (0, 0)),
        ],
        out_specs=[
            pl.BlockSpec((G, H), lambda i: (0, 0)),
            pl.BlockSpec((G, H), lambda i: (0, 0)),
            pl.BlockSpec((G, H), lambda i: (0, 0)),
            pl.BlockSpec((G, 2), lambda i: (0, 0)),
        ],
        out_shape=[
            jax.ShapeDtypeStruct((G, H), jnp.float32),
            jax.ShapeDtypeStruct((G, H), jnp.float32),
            jax.ShapeDtypeStruct((G, H), jnp.float32),
            jax.ShapeDtypeStruct((G, 2), jnp.float32),
        ],
    )(p, degp4, node_weight.reshape(N, 1), graph_ids.reshape(N, 1), W_conv)

    out = pl.pallas_call(
        _fin_body,
        out_shape=jax.ShapeDtypeStruct((G, OUT), jnp.float32),
    )(s1, s2, t1, c, gn_gamma.reshape(1, H), gn_beta.reshape(1, H),
      gn_alpha.reshape(1, H), W_lin, W_cls)
    return out


# speedup vs baseline: 10.6663x; 1.6404x over previous
import functools

import jax
import jax.numpy as jnp
from jax import lax
from jax.experimental import pallas as pl
from jax.experimental.pallas import tpu as pltpu
from jax.experimental.pallas import tpu_sc as plsc

N = 10000
E = 320000
F = 128
H = 256
G = 64
OUT = 10

NC = 2
NS = 16
NW = NC * NS
CE = 128
NCHUNK = E // CE
NP = 10240
ZB = NP // NS

NPAIR = NCHUNK // 2
PB = NPAIR // NW
PR = NPAIR % NW
NQ = NCHUNK // 4
QB = NQ // NW
QR = NQ % NW

CE2 = 2 * CE


def _deg_body(src_hbm, dst_hbm, out_hbm, sa, da, sb, db, ones, zbuf,
              acc_src, acc_dst, sem_sa, sem_da, sem_sb, sem_db):
    cid = lax.axis_index("c")
    sid = lax.axis_index("s")
    wid = sid * NC + cid
    for i in range(CE // 16):
        ones[pl.ds(16 * i, 16)] = jnp.ones((16,), jnp.float32)
    for i in range(ZB // 16):
        zbuf[pl.ds(16 * i, 16)] = jnp.zeros((16,), jnp.float32)
    pltpu.sync_copy(zbuf, acc_src.at[pl.ds(sid * ZB, ZB)])
    pltpu.sync_copy(zbuf, acc_dst.at[pl.ds(sid * ZB, ZB)])
    plsc.subcore_barrier()

    nq = QB + jnp.where(wid < QR, 1, 0)

    eb0 = 4 * wid * CE
    pltpu.async_copy(src_hbm.at[pl.ds(eb0, CE2)], sa, sem_sa)
    pltpu.async_copy(dst_hbm.at[pl.ds(eb0, CE2)], da, sem_da)

    def body(q, carry):
        eb = 4 * (wid + q * NW) * CE
        ebn = 4 * (wid + (q + 1) * NW) * CE
        pltpu.make_async_copy(src_hbm.at[pl.ds(eb, CE2)], sa, sem_sa).wait()
        pltpu.make_async_copy(dst_hbm.at[pl.ds(eb, CE2)], da, sem_da).wait()
        pltpu.async_copy(src_hbm.at[pl.ds(eb + CE2, CE2)], sb, sem_sb)
        pltpu.async_copy(dst_hbm.at[pl.ds(eb + CE2, CE2)], db, sem_db)
        pltpu.sync_copy(ones, acc_src.at[sa.at[pl.ds(0, CE)]], add=True)
        pltpu.sync_copy(ones, acc_src.at[sa.at[pl.ds(CE, CE)]], add=True)
        pltpu.sync_copy(ones, acc_dst.at[da.at[pl.ds(0, CE)]], add=True)
        pltpu.sync_copy(ones, acc_dst.at[da.at[pl.ds(CE, CE)]], add=True)
        pltpu.make_async_copy(src_hbm.at[pl.ds(eb + CE2, CE2)], sb,
                              sem_sb).wait()
        pltpu.make_async_copy(dst_hbm.at[pl.ds(eb + CE2, CE2)], db,
                              sem_db).wait()

        @pl.when(q + 1 < nq)
        def _next():
            pltpu.async_copy(src_hbm.at[pl.ds(ebn, CE2)], sa, sem_sa)
            pltpu.async_copy(dst_hbm.at[pl.ds(ebn, CE2)], da, sem_da)

        pltpu.sync_copy(ones, acc_src.at[sb.at[pl.ds(0, CE)]], add=True)
        pltpu.sync_copy(ones, acc_src.at[sb.at[pl.ds(CE, CE)]], add=True)
        pltpu.sync_copy(ones, acc_dst.at[db.at[pl.ds(0, CE)]], add=True)
        pltpu.sync_copy(ones, acc_dst.at[db.at[pl.ds(CE, CE)]], add=True)
        return carry

    lax.fori_loop(0, nq, body, 0)
    plsc.subcore_barrier()
    pltpu.sync_copy(acc_src.at[pl.ds(sid * ZB, ZB)],
                    out_hbm.at[cid, 0, pl.ds(sid * ZB, ZB)])
    pltpu.sync_copy(acc_dst.at[pl.ds(sid * ZB, ZB)],
                    out_hbm.at[cid, 1, pl.ds(sid * ZB, ZB)])


def _spmm_body(src_hbm, dst_hbm, ew_hbm, xs_hbm, out_hbm,
               sbuf, dbuf, ewbuf, db2, ewb2, rows0, rows1, acc,
               sem_g0, sem_g1):
    cid = lax.axis_index("c")
    sid = lax.axis_index("s")
    wid = sid * NC + cid

    def zfill(i, carry):
        for k in range(F // 16):
            rows0[i, pl.ds(16 * k, 16)] = jnp.zeros((16,), jnp.float32)
        return carry

    lax.fori_loop(0, CE, zfill, 0)
    for j in range(ZB // CE):
        pltpu.sync_copy(rows0, acc.at[pl.ds(sid * ZB + j * CE, CE), :])
    plsc.subcore_barrier()

    npairs = PB + jnp.where(wid < PR, 1, 0)

    def escale(rows, ewb, e16, lo):
        wv = ewb[pl.ds(lo + e16 * 16, 16)]
        for j in range(16):
            wj = wv.at[jnp.full((16,), j, jnp.int32)].get(
                mode='promise_in_bounds')
            for k in range(F // 16):
                sl = pl.ds(16 * k, 16)
                rows[e16 * 16 + j, sl] = rows[e16 * 16 + j, sl] * wj

    eb0 = 2 * wid * CE
    pltpu.sync_copy(src_hbm.at[pl.ds(eb0, CE2)], sbuf)
    pltpu.sync_copy(dst_hbm.at[pl.ds(eb0, CE2)], dbuf)
    pltpu.sync_copy(ew_hbm.at[pl.ds(eb0, CE2)], ewbuf)
    pltpu.async_copy(xs_hbm.at[sbuf.at[pl.ds(0, CE)]], rows0, sem_g0)

    def body(j, carry):
        ebn = 2 * (wid + (j + 1) * NW) * CE
        pltpu.async_copy(xs_hbm.at[sbuf.at[pl.ds(CE, CE)]], rows1, sem_g1)
        pltpu.make_async_copy(xs_hbm.at[sbuf.at[pl.ds(0, CE)]], rows0,
                              sem_g0).wait()

        def esc_a(e16, c2):
            escale(rows0, ewbuf, e16, 0)
            return c2

        lax.fori_loop(0, CE // 16, esc_a, 0)
        pltpu.sync_copy(rows0, acc.at[dbuf.at[pl.ds(0, CE)]], add=True)
        pltpu.make_async_copy(xs_hbm.at[sbuf.at[pl.ds(CE, CE)]], rows1,
                              sem_g1).wait()
        for k in range(CE // 16):
            sl = pl.ds(16 * k, 16)
            slb = pl.ds(CE + 16 * k, 16)
            db2[sl] = dbuf[slb]
            ewb2[sl] = ewbuf[slb]

        @pl.when(j + 1 < npairs)
        def _next():
            pltpu.sync_copy(src_hbm.at[pl.ds(ebn, CE2)], sbuf)
            pltpu.sync_copy(dst_hbm.at[pl.ds(ebn, CE2)], dbuf)
            pltpu.sync_copy(ew_hbm.at[pl.ds(ebn, CE2)], ewbuf)
            pltpu.async_copy(xs_hbm.at[sbuf.at[pl.ds(0, CE)]], rows0, sem_g0)

        def esc_b(e16, c2):
            escale(rows1, ewb2, e16, 0)
            return c2

        lax.fori_loop(0, CE // 16, esc_b, 0)
        pltpu.sync_copy(rows1, acc.at[db2], add=True)
        return carry

    lax.fori_loop(0, npairs, body, 0)
    plsc.subcore_barrier()
    for j in range(ZB // 64):
        pltpu.sync_copy(acc.at[pl.ds(sid * ZB + j * 64, 64), :],
                        out_hbm.at[cid, pl.ds(sid * ZB + j * 64, 64), :])


def _xs_body(x_ref, degp_ref, xs_ref):
    dout = degp_ref[0, :N, :] + degp_ref[2, :N, :]
    scale = lax.rsqrt(jnp.maximum(dout, 1.0))
    xs_ref[...] = x_ref[...] * scale


def _leaky(z):
    return jnp.where(z >= 0, z, 0.01 * z)


_HI = lax.Precision.HIGHEST


BN = 2000


def _mom_body(p_ref, degp_ref, nw_ref, gid_ref, wc_ref,
              s1_ref, s2_ref, t1_ref, c_ref):
    f32 = jnp.float32
    agg = p_ref[0] + p_ref[1]
    din = degp_ref[1] + degp_ref[3]
    sin = lax.rsqrt(jnp.maximum(din, 1.0))
    h = jnp.dot(agg, wc_ref[...], preferred_element_type=f32,
                precision=_HI) * sin
    h = _leaky(h)

    ids = gid_ref[...]
    iota_g = lax.broadcasted_iota(jnp.int32, (BN, G), 1)
    m = (ids == iota_g).astype(f32)

    seg = lambda v: lax.dot_general(
        m, v, (((0,), (0,)), ((), ())), preferred_element_type=f32,
        precision=_HI)

    nw = nw_ref[...]
    onw = jnp.concatenate([jnp.ones((BN, 1), f32), nw], axis=1)

    @pl.when(pl.program_id(0) == 0)
    def _init():
        s1_ref[...] = jnp.zeros_like(s1_ref)
        s2_ref[...] = jnp.zeros_like(s2_ref)
        t1_ref[...] = jnp.zeros_like(t1_ref)
        c_ref[...] = jnp.zeros_like(c_ref)

    s1_ref[...] += seg(h)
    s2_ref[...] += seg(h * h)
    t1_ref[...] += seg(nw * h)
    c_ref[...] += seg(onw)


def _fin_body(s1_ref, s2_ref, t1_ref, c_ref, gg_ref, gb_ref, ga_ref,
              wl_ref, wcls_ref, out_ref):
    f32 = jnp.float32
    cnt = jnp.maximum(c_ref[:, 0:1], 1.0)
    wg = c_ref[:, 1:2]
    inv = 1.0 / cnt
    a = ga_ref[...]
    mean = s1_ref[...] * inv
    var = s2_ref[...] * inv - (2.0 * a - a * a) * mean * mean
    hscale = gg_ref[...] * lax.rsqrt(var + 1e-5)
    r = (hscale * (t1_ref[...] - a * mean * wg) + gb_ref[...] * wg) * inv
    r2 = _leaky(jnp.dot(r, wl_ref[...], preferred_element_type=f32,
                        precision=_HI))
    mu = jnp.mean(r2, axis=1, keepdims=True)
    v = jnp.mean((r2 - mu) ** 2, axis=1, keepdims=True)
    rn = (r2 - mu) * lax.rsqrt(v + 1e-5)
    out_ref[...] = jnp.dot(rn, wcls_ref[...], preferred_element_type=f32,
                           precision=_HI)


@functools.lru_cache(maxsize=None)
def _build_sc_kernels():
    mesh = plsc.VectorSubcoreMesh(
        core_axis_name="c", subcore_axis_name="s",
        num_cores=NC, num_subcores=NS)
    deg = pl.kernel(
        _deg_body,
        out_type=jax.ShapeDtypeStruct((NC, 2, NP), jnp.float32),
        mesh=mesh,
        scratch_types=[
            pltpu.VMEM((CE2,), jnp.int32),
            pltpu.VMEM((CE2,), jnp.int32),
            pltpu.VMEM((CE2,), jnp.int32),
            pltpu.VMEM((CE2,), jnp.int32),
            pltpu.VMEM((CE,), jnp.float32),
            pltpu.VMEM((ZB,), jnp.float32),
            pltpu.VMEM_SHARED((NP,), jnp.float32),
            pltpu.VMEM_SHARED((NP,), jnp.float32),
            pltpu.SemaphoreType.DMA,
            pltpu.SemaphoreType.DMA,
            pltpu.SemaphoreType.DMA,
            pltpu.SemaphoreType.DMA,
        ],
    )
    spmm = pl.kernel(
        _spmm_body,
        out_type=jax.ShapeDtypeStruct((NC, NP, F), jnp.float32),
        mesh=mesh,
        scratch_types=[
            pltpu.VMEM((CE2,), jnp.int32),
            pltpu.VMEM((CE2,), jnp.int32),
            pltpu.VMEM((CE2,), jnp.float32),
            pltpu.VMEM((CE,), jnp.int32),
            pltpu.VMEM((CE,), jnp.float32),
            pltpu.VMEM((CE, F), jnp.float32),
            pltpu.VMEM((CE, F), jnp.float32),
            pltpu.VMEM_SHARED((NP, F), jnp.float32),
            pltpu.SemaphoreType.DMA,
            pltpu.SemaphoreType.DMA,
        ],
    )
    return deg, spmm


def kernel(x, edge_index, edge_weight, node_weight, graph_ids, W_conv,
           gn_gamma, gn_beta, gn_alpha, W_lin, W_cls):
    src = edge_index[0]
    dst = edge_index[1]
    _deg_kernel, _spmm_kernel = _build_sc_kernels()

    degp = _deg_kernel(src, dst)
    degp4 = degp.reshape(4, NP, 1)

    xs = pl.pallas_call(
        _xs_body,
        out_shape=jax.ShapeDtypeStruct((N, F), jnp.float32),
    )(x, degp4)

    p = _spmm_kernel(src, dst, edge_weight, xs)

    s1, s2, t1, c = pl.pallas_call(
        _mom_body,
        grid=(N // BN,),
        in_specs=[
            pl.BlockSpec((2, BN, F), lambda i: (0, i, 0)),
            pl.BlockSpec((4, BN, 1), lambda i: (0, i, 0)),
            pl.BlockSpec((BN, 1), lambda i: (i, 0)),
            pl.BlockSpec((BN, 1), lambda i: (i, 0)),
            pl.BlockSpec((F, H), lambda i: (0, 0)),
        ],
        out_specs=[
            pl.BlockSpec((G, H), lambda i: (0, 0)),
            pl.BlockSpec((G, H), lambda i: (0, 0)),
            pl.BlockSpec((G, H), lambda i: (0, 0)),
            pl.BlockSpec((G, 2), lambda i: (0, 0)),
        ],
        out_shape=[
            jax.ShapeDtypeStruct((G, H), jnp.float32),
            jax.ShapeDtypeStruct((G, H), jnp.float32),
            jax.ShapeDtypeStruct((G, H), jnp.float32),
            jax.ShapeDtypeStruct((G, 2), jnp.float32),
        ],
    )(p, degp4, node_weight.reshape(N, 1), graph_ids.reshape(N, 1), W_conv)

    out = pl.pallas_call(
        _fin_body,
        out_shape=jax.ShapeDtypeStruct((G, OUT), jnp.float32),
    )(s1, s2, t1, c, gn_gamma.reshape(1, H), gn_beta.reshape(1, H),
      gn_alpha.reshape(1, H), W_lin, W_cls)
    return out


# speedup vs baseline: 11.8960x; 1.1153x over previous
import functools

import jax
import jax.numpy as jnp
from jax import lax
from jax.experimental import pallas as pl
from jax.experimental.pallas import tpu as pltpu
from jax.experimental.pallas import tpu_sc as plsc

N = 10000
E = 320000
F = 128
H = 256
G = 64
OUT = 10

NC = 2
NS = 16
NW = NC * NS
CE = 128
NCHUNK = E // CE
NP = 10240
ZB = NP // NS

NPAIR = NCHUNK // 2
PB = NPAIR // NW
PR = NPAIR % NW
NQ = NCHUNK // 4
QB = NQ // NW
QR = NQ % NW

CE2 = 2 * CE


def _deg_body(src_hbm, dst_hbm, out_hbm, sa, da, sb, db, ones, zbuf,
              acc_src, acc_dst, sem_sa, sem_da, sem_sb, sem_db):
    cid = lax.axis_index("c")
    sid = lax.axis_index("s")
    wid = sid * NC + cid
    for i in range(CE // 16):
        ones[pl.ds(16 * i, 16)] = jnp.ones((16,), jnp.float32)
    for i in range(ZB // 16):
        zbuf[pl.ds(16 * i, 16)] = jnp.zeros((16,), jnp.float32)
    pltpu.sync_copy(zbuf, acc_src.at[pl.ds(sid * ZB, ZB)])
    pltpu.sync_copy(zbuf, acc_dst.at[pl.ds(sid * ZB, ZB)])
    plsc.subcore_barrier()

    nq = QB + jnp.where(wid < QR, 1, 0)

    eb0 = 4 * wid * CE
    pltpu.async_copy(src_hbm.at[pl.ds(eb0, CE2)], sa, sem_sa)
    pltpu.async_copy(dst_hbm.at[pl.ds(eb0, CE2)], da, sem_da)

    def body(q, carry):
        eb = 4 * (wid + q * NW) * CE
        ebn = 4 * (wid + (q + 1) * NW) * CE
        pltpu.make_async_copy(src_hbm.at[pl.ds(eb, CE2)], sa, sem_sa).wait()
        pltpu.make_async_copy(dst_hbm.at[pl.ds(eb, CE2)], da, sem_da).wait()
        pltpu.async_copy(src_hbm.at[pl.ds(eb + CE2, CE2)], sb, sem_sb)
        pltpu.async_copy(dst_hbm.at[pl.ds(eb + CE2, CE2)], db, sem_db)
        pltpu.sync_copy(ones, acc_src.at[sa.at[pl.ds(0, CE)]], add=True)
        pltpu.sync_copy(ones, acc_src.at[sa.at[pl.ds(CE, CE)]], add=True)
        pltpu.sync_copy(ones, acc_dst.at[da.at[pl.ds(0, CE)]], add=True)
        pltpu.sync_copy(ones, acc_dst.at[da.at[pl.ds(CE, CE)]], add=True)
        pltpu.make_async_copy(src_hbm.at[pl.ds(eb + CE2, CE2)], sb,
                              sem_sb).wait()
        pltpu.make_async_copy(dst_hbm.at[pl.ds(eb + CE2, CE2)], db,
                              sem_db).wait()

        @pl.when(q + 1 < nq)
        def _next():
            pltpu.async_copy(src_hbm.at[pl.ds(ebn, CE2)], sa, sem_sa)
            pltpu.async_copy(dst_hbm.at[pl.ds(ebn, CE2)], da, sem_da)

        pltpu.sync_copy(ones, acc_src.at[sb.at[pl.ds(0, CE)]], add=True)
        pltpu.sync_copy(ones, acc_src.at[sb.at[pl.ds(CE, CE)]], add=True)
        pltpu.sync_copy(ones, acc_dst.at[db.at[pl.ds(0, CE)]], add=True)
        pltpu.sync_copy(ones, acc_dst.at[db.at[pl.ds(CE, CE)]], add=True)
        return carry

    lax.fori_loop(0, nq, body, 0)
    plsc.subcore_barrier()
    pltpu.sync_copy(acc_src.at[pl.ds(sid * ZB, ZB)],
                    out_hbm.at[cid, 0, pl.ds(sid * ZB, ZB)])
    pltpu.sync_copy(acc_dst.at[pl.ds(sid * ZB, ZB)],
                    out_hbm.at[cid, 1, pl.ds(sid * ZB, ZB)])


_RSQRT_MAGIC = 0x5F3759DF


def _newton_rsqrt(m):
    bi = lax.bitcast_convert_type(m, jnp.int32)
    y = lax.bitcast_convert_type(
        jnp.full((16,), _RSQRT_MAGIC, jnp.int32)
        - lax.shift_right_arithmetic(bi, jnp.full((16,), 1, jnp.int32)),
        jnp.float32)
    half = m * (-0.5)
    for _ in range(3):
        y = y * (half * y * y + 1.5)
    return y


def _spmm_body(src_hbm, dst_hbm, ew_hbm, x_hbm, degp_hbm, out_hbm,
               sbuf, dbuf, ewbuf, db2, ewb2, rsga, rsgb, hb0, hb1, rsl,
               rows0, rows1, acc, rs_sp,
               sem_g0, sem_g1, sem_r0, sem_r1):
    cid = lax.axis_index("c")
    sid = lax.axis_index("s")
    wid = sid * NC + cid

    pltpu.sync_copy(degp_hbm.at[0, 0, pl.ds(sid * ZB, ZB)], hb0)
    pltpu.sync_copy(degp_hbm.at[1, 0, pl.ds(sid * ZB, ZB)], hb1)

    def rsloop(i, carry):
        sl = pl.ds(i * 16, 16)
        m = jnp.maximum(hb0[sl] + hb1[sl], 1.0)
        rsl[sl] = _newton_rsqrt(m)
        return carry

    lax.fori_loop(0, ZB // 16, rsloop, 0)
    pltpu.sync_copy(rsl, rs_sp.at[pl.ds(sid * ZB, ZB)])

    def zfill(i, carry):
        for k in range(F // 16):
            rows0[i, pl.ds(16 * k, 16)] = jnp.zeros((16,), jnp.float32)
        return carry

    lax.fori_loop(0, CE, zfill, 0)
    for j in range(ZB // CE):
        pltpu.sync_copy(rows0, acc.at[pl.ds(sid * ZB + j * CE, CE), :])
    plsc.subcore_barrier()

    npairs = PB + jnp.where(wid < PR, 1, 0)

    def escale(rows, ewb, e16, lo):
        wv = ewb[pl.ds(lo + e16 * 16, 16)]
        for j in range(16):
            wj = wv.at[jnp.full((16,), j, jnp.int32)].get(
                mode='promise_in_bounds')
            for k in range(F // 16):
                sl = pl.ds(16 * k, 16)
                rows[e16 * 16 + j, sl] = rows[e16 * 16 + j, sl] * wj

    eb0 = 2 * wid * CE
    pltpu.sync_copy(src_hbm.at[pl.ds(eb0, CE2)], sbuf)
    pltpu.sync_copy(dst_hbm.at[pl.ds(eb0, CE2)], dbuf)
    pltpu.sync_copy(ew_hbm.at[pl.ds(eb0, CE2)], ewbuf)
    pltpu.async_copy(x_hbm.at[sbuf.at[pl.ds(0, CE)]], rows0, sem_g0)
    pltpu.async_copy(rs_sp.at[sbuf.at[pl.ds(0, CE)]], rsga, sem_r0)

    def body(j, carry):
        ebn = 2 * (wid + (j + 1) * NW) * CE
        pltpu.async_copy(x_hbm.at[sbuf.at[pl.ds(CE, CE)]], rows1, sem_g1)
        pltpu.async_copy(rs_sp.at[sbuf.at[pl.ds(CE, CE)]], rsgb, sem_r1)
        pltpu.make_async_copy(x_hbm.at[sbuf.at[pl.ds(0, CE)]], rows0,
                              sem_g0).wait()
        pltpu.make_async_copy(rs_sp.at[sbuf.at[pl.ds(0, CE)]], rsga,
                              sem_r0).wait()
        for k in range(CE // 16):
            sl = pl.ds(16 * k, 16)
            ewbuf[sl] = ewbuf[sl] * rsga[sl]

        def esc_a(e16, c2):
            escale(rows0, ewbuf, e16, 0)
            return c2

        lax.fori_loop(0, CE // 16, esc_a, 0)
        pltpu.sync_copy(rows0, acc.at[dbuf.at[pl.ds(0, CE)]], add=True)
        pltpu.make_async_copy(x_hbm.at[sbuf.at[pl.ds(CE, CE)]], rows1,
                              sem_g1).wait()
        pltpu.make_async_copy(rs_sp.at[sbuf.at[pl.ds(CE, CE)]], rsgb,
                              sem_r1).wait()
        for k in range(CE // 16):
            sl = pl.ds(16 * k, 16)
            slb = pl.ds(CE + 16 * k, 16)
            db2[sl] = dbuf[slb]
            ewb2[sl] = ewbuf[slb] * rsgb[sl]

        @pl.when(j + 1 < npairs)
        def _next():
            pltpu.sync_copy(src_hbm.at[pl.ds(ebn, CE2)], sbuf)
            pltpu.sync_copy(dst_hbm.at[pl.ds(ebn, CE2)], dbuf)
            pltpu.sync_copy(ew_hbm.at[pl.ds(ebn, CE2)], ewbuf)
            pltpu.async_copy(x_hbm.at[sbuf.at[pl.ds(0, CE)]], rows0, sem_g0)
            pltpu.async_copy(rs_sp.at[sbuf.at[pl.ds(0, CE)]], rsga, sem_r0)

        def esc_b(e16, c2):
            escale(rows1, ewb2, e16, 0)
            return c2

        lax.fori_loop(0, CE // 16, esc_b, 0)
        pltpu.sync_copy(rows1, acc.at[db2], add=True)
        return carry

    lax.fori_loop(0, npairs, body, 0)
    plsc.subcore_barrier()
    for j in range(ZB // 64):
        pltpu.sync_copy(acc.at[pl.ds(sid * ZB + j * 64, 64), :],
                        out_hbm.at[cid, pl.ds(sid * ZB + j * 64, 64), :])


def _leaky(z):
    return jnp.where(z >= 0, z, 0.01 * z)


_HI = lax.Precision.HIGHEST


BN = 2000


def _mom_body(p_ref, degp_ref, nw_ref, gid_ref, wc_ref,
              s1_ref, s2_ref, t1_ref, c_ref):
    f32 = jnp.float32
    agg = p_ref[0] + p_ref[1]
    din = degp_ref[1] + degp_ref[3]
    sin = lax.rsqrt(jnp.maximum(din, 1.0))
    h = jnp.dot(agg, wc_ref[...], preferred_element_type=f32,
                precision=_HI) * sin
    h = _leaky(h)

    ids = gid_ref[...]
    iota_g = lax.broadcasted_iota(jnp.int32, (BN, G), 1)
    m = (ids == iota_g).astype(f32)

    seg = lambda v: lax.dot_general(
        m, v, (((0,), (0,)), ((), ())), preferred_element_type=f32,
        precision=_HI)

    nw = nw_ref[...]
    onw = jnp.concatenate([jnp.ones((BN, 1), f32), nw], axis=1)

    @pl.when(pl.program_id(0) == 0)
    def _init():
        s1_ref[...] = jnp.zeros_like(s1_ref)
        s2_ref[...] = jnp.zeros_like(s2_ref)
        t1_ref[...] = jnp.zeros_like(t1_ref)
        c_ref[...] = jnp.zeros_like(c_ref)

    s1_ref[...] += seg(h)
    s2_ref[...] += seg(h * h)
    t1_ref[...] += seg(nw * h)
    c_ref[...] += seg(onw)


def _fin_body(s1_ref, s2_ref, t1_ref, c_ref, gg_ref, gb_ref, ga_ref,
              wl_ref, wcls_ref, out_ref):
    f32 = jnp.float32
    cnt = jnp.maximum(c_ref[:, 0:1], 1.0)
    wg = c_ref[:, 1:2]
    inv = 1.0 / cnt
    a = ga_ref[...]
    mean = s1_ref[...] * inv
    var = s2_ref[...] * inv - (2.0 * a - a * a) * mean * mean
    hscale = gg_ref[...] * lax.rsqrt(var + 1e-5)
    r = (hscale * (t1_ref[...] - a * mean * wg) + gb_ref[...] * wg) * inv
    r2 = _leaky(jnp.dot(r, wl_ref[...], preferred_element_type=f32,
                        precision=_HI))
    mu = jnp.mean(r2, axis=1, keepdims=True)
    v = jnp.mean((r2 - mu) ** 2, axis=1, keepdims=True)
    rn = (r2 - mu) * lax.rsqrt(v + 1e-5)
    out_ref[...] = jnp.dot(rn, wcls_ref[...], preferred_element_type=f32,
                           precision=_HI)


@functools.lru_cache(maxsize=None)
def _build_sc_kernels():
    mesh = plsc.VectorSubcoreMesh(
        core_axis_name="c", subcore_axis_name="s",
        num_cores=NC, num_subcores=NS)
    deg = pl.kernel(
        _deg_body,
        out_type=jax.ShapeDtypeStruct((NC, 2, NP), jnp.float32),
        mesh=mesh,
        scratch_types=[
            pltpu.VMEM((CE2,), jnp.int32),
            pltpu.VMEM((CE2,), jnp.int32),
            pltpu.VMEM((CE2,), jnp.int32),
            pltpu.VMEM((CE2,), jnp.int32),
            pltpu.VMEM((CE,), jnp.float32),
            pltpu.VMEM((ZB,), jnp.float32),
            pltpu.VMEM_SHARED((NP,), jnp.float32),
            pltpu.VMEM_SHARED((NP,), jnp.float32),
            pltpu.SemaphoreType.DMA,
            pltpu.SemaphoreType.DMA,
            pltpu.SemaphoreType.DMA,
            pltpu.SemaphoreType.DMA,
        ],
    )
    spmm = pl.kernel(
        _spmm_body,
        out_type=jax.ShapeDtypeStruct((NC, NP, F), jnp.float32),
        mesh=mesh,
        scratch_types=[
            pltpu.VMEM((CE2,), jnp.int32),
            pltpu.VMEM((CE2,), jnp.int32),
            pltpu.VMEM((CE2,), jnp.float32),
            pltpu.VMEM((CE,), jnp.int32),
            pltpu.VMEM((CE,), jnp.float32),
            pltpu.VMEM((CE,), jnp.float32),
            pltpu.VMEM((CE,), jnp.float32),
            pltpu.VMEM((ZB,), jnp.float32),
            pltpu.VMEM((ZB,), jnp.float32),
            pltpu.VMEM((ZB,), jnp.float32),
            pltpu.VMEM((CE, F), jnp.float32),
            pltpu.VMEM((CE, F), jnp.float32),
            pltpu.VMEM_SHARED((NP, F), jnp.float32),
            pltpu.VMEM_SHARED((NP,), jnp.float32),
            pltpu.SemaphoreType.DMA,
            pltpu.SemaphoreType.DMA,
            pltpu.SemaphoreType.DMA,
            pltpu.SemaphoreType.DMA,
        ],
    )
    return deg, spmm


def kernel(x, edge_index, edge_weight, node_weight, graph_ids, W_conv,
           gn_gamma, gn_beta, gn_alpha, W_lin, W_cls):
    src = edge_index[0]
    dst = edge_index[1]
    _deg_kernel, _spmm_kernel = _build_sc_kernels()

    degp = _deg_kernel(src, dst)
    degp4 = degp.reshape(4, NP, 1)

    p = _spmm_kernel(src, dst, edge_weight, x, degp)

    s1, s2, t1, c = pl.pallas_call(
        _mom_body,
        grid=(N // BN,),
        in_specs=[
            pl.BlockSpec((2, BN, F), lambda i: (0, i, 0)),
            pl.BlockSpec((4, BN, 1), lambda i: (0, i, 0)),
            pl.BlockSpec((BN, 1), lambda i: (i, 0)),
            pl.BlockSpec((BN, 1), lambda i: (i, 0)),
            pl.BlockSpec((F, H), lambda i: (0, 0)),
        ],
        out_specs=[
            pl.BlockSpec((G, H), lambda i: (0, 0)),
            pl.BlockSpec((G, H), lambda i: (0, 0)),
            pl.BlockSpec((G, H), lambda i: (0, 0)),
            pl.BlockSpec((G, 2), lambda i: (0, 0)),
        ],
        out_shape=[
            jax.ShapeDtypeStruct((G, H), jnp.float32),
            jax.ShapeDtypeStruct((G, H), jnp.float32),
            jax.ShapeDtypeStruct((G, H), jnp.float32),
            jax.ShapeDtypeStruct((G, 2), jnp.float32),
        ],
    )(p, degp4, node_weight.reshape(N, 1), graph_ids.reshape(N, 1), W_conv)

    out = pl.pallas_call(
        _fin_body,
        out_shape=jax.ShapeDtypeStruct((G, OUT), jnp.float32),
    )(s1, s2, t1, c, gn_gamma.reshape(1, H), gn_beta.reshape(1, H),
      gn_alpha.reshape(1, H), W_lin, W_cls)
    return out


# speedup vs baseline: 11.9764x; 1.0068x over previous
import functools

import jax
import jax.numpy as jnp
from jax import lax
from jax.experimental import pallas as pl
from jax.experimental.pallas import tpu as pltpu
from jax.experimental.pallas import tpu_sc as plsc

N = 10000
E = 320000
F = 128
H = 256
G = 64
OUT = 10

NC = 2
NS = 16
NW = NC * NS
CE = 128
NCHUNK = E // CE
NP = 10240
ZB = NP // NS

NPAIR = NCHUNK // 2
PB = NPAIR // NW
PR = NPAIR % NW
NQ = NCHUNK // 4
QB = NQ // NW
QR = NQ % NW

CE2 = 2 * CE


def _deg_body(src_hbm, dst_hbm, out_hbm, sa, da, sb, db, ones, zbuf,
              acc_src, acc_dst, sem_sa, sem_da, sem_sb, sem_db):
    cid = lax.axis_index("c")
    sid = lax.axis_index("s")
    wid = sid * NC + cid
    for i in range(CE // 16):
        ones[pl.ds(16 * i, 16)] = jnp.ones((16,), jnp.float32)
    for i in range(ZB // 16):
        zbuf[pl.ds(16 * i, 16)] = jnp.zeros((16,), jnp.float32)
    pltpu.sync_copy(zbuf, acc_src.at[pl.ds(sid * ZB, ZB)])
    pltpu.sync_copy(zbuf, acc_dst.at[pl.ds(sid * ZB, ZB)])
    plsc.subcore_barrier()

    nq = QB + jnp.where(wid < QR, 1, 0)

    eb0 = 4 * wid * CE
    pltpu.async_copy(src_hbm.at[pl.ds(eb0, CE2)], sa, sem_sa)
    pltpu.async_copy(dst_hbm.at[pl.ds(eb0, CE2)], da, sem_da)

    def body(q, carry):
        eb = 4 * (wid + q * NW) * CE
        ebn = 4 * (wid + (q + 1) * NW) * CE
        pltpu.make_async_copy(src_hbm.at[pl.ds(eb, CE2)], sa, sem_sa).wait()
        pltpu.make_async_copy(dst_hbm.at[pl.ds(eb, CE2)], da, sem_da).wait()
        pltpu.async_copy(src_hbm.at[pl.ds(eb + CE2, CE2)], sb, sem_sb)
        pltpu.async_copy(dst_hbm.at[pl.ds(eb + CE2, CE2)], db, sem_db)
        pltpu.sync_copy(ones, acc_src.at[sa.at[pl.ds(0, CE)]], add=True)
        pltpu.sync_copy(ones, acc_src.at[sa.at[pl.ds(CE, CE)]], add=True)
        pltpu.sync_copy(ones, acc_dst.at[da.at[pl.ds(0, CE)]], add=True)
        pltpu.sync_copy(ones, acc_dst.at[da.at[pl.ds(CE, CE)]], add=True)
        pltpu.make_async_copy(src_hbm.at[pl.ds(eb + CE2, CE2)], sb,
                              sem_sb).wait()
        pltpu.make_async_copy(dst_hbm.at[pl.ds(eb + CE2, CE2)], db,
                              sem_db).wait()

        @pl.when(q + 1 < nq)
        def _next():
            pltpu.async_copy(src_hbm.at[pl.ds(ebn, CE2)], sa, sem_sa)
            pltpu.async_copy(dst_hbm.at[pl.ds(ebn, CE2)], da, sem_da)

        pltpu.sync_copy(ones, acc_src.at[sb.at[pl.ds(0, CE)]], add=True)
        pltpu.sync_copy(ones, acc_src.at[sb.at[pl.ds(CE, CE)]], add=True)
        pltpu.sync_copy(ones, acc_dst.at[db.at[pl.ds(0, CE)]], add=True)
        pltpu.sync_copy(ones, acc_dst.at[db.at[pl.ds(CE, CE)]], add=True)
        return carry

    lax.fori_loop(0, nq, body, 0)
    plsc.subcore_barrier()
    pltpu.sync_copy(acc_src.at[pl.ds(sid * ZB, ZB)],
                    out_hbm.at[cid, 0, pl.ds(sid * ZB, ZB)])
    pltpu.sync_copy(acc_dst.at[pl.ds(sid * ZB, ZB)],
                    out_hbm.at[cid, 1, pl.ds(sid * ZB, ZB)])


_RSQRT_MAGIC = 0x5F3759DF


def _newton_rsqrt(m):
    bi = lax.bitcast_convert_type(m, jnp.int32)
    y = lax.bitcast_convert_type(
        jnp.full((16,), _RSQRT_MAGIC, jnp.int32)
        - lax.shift_right_arithmetic(bi, jnp.full((16,), 1, jnp.int32)),
        jnp.float32)
    half = m * (-0.5)
    for _ in range(3):
        y = y * (half * y * y + 1.5)
    return y


def _spmm_body(src_hbm, dst_hbm, ew_hbm, x_hbm, degp_hbm, out_hbm,
               sbuf, dbuf, ewbuf, db2, ewb2, rsga, rsgb, hb0, hb1, rsl,
               rows0, rows1, acc, rs_sp,
               sem_g0, sem_g1, sem_r0, sem_r1):
    cid = lax.axis_index("c")
    sid = lax.axis_index("s")
    wid = sid * NC + cid

    pltpu.sync_copy(degp_hbm.at[0, 0, pl.ds(sid * ZB, ZB)], hb0)
    pltpu.sync_copy(degp_hbm.at[1, 0, pl.ds(sid * ZB, ZB)], hb1)

    def rsloop(i, carry):
        sl = pl.ds(i * 16, 16)
        m = jnp.maximum(hb0[sl] + hb1[sl], 1.0)
        rsl[sl] = _newton_rsqrt(m)
        return carry

    lax.fori_loop(0, ZB // 16, rsloop, 0)
    pltpu.sync_copy(rsl, rs_sp.at[pl.ds(sid * ZB, ZB)])

    def zfill(i, carry):
        for k in range(F // 16):
            rows0[i, pl.ds(16 * k, 16)] = jnp.zeros((16,), jnp.float32)
        return carry

    lax.fori_loop(0, CE, zfill, 0)
    for j in range(ZB // CE):
        pltpu.sync_copy(rows0, acc.at[pl.ds(sid * ZB + j * CE, CE), :])
    plsc.subcore_barrier()

    npairs = PB + jnp.where(wid < PR, 1, 0)

    def escale(rows, ewb, e16, lo):
        wv = ewb[pl.ds(lo + e16 * 16, 16)]
        for j in range(16):
            wj = wv.at[jnp.full((16,), j, jnp.int32)].get(
                mode='promise_in_bounds')
            for k in range(F // 16):
                sl = pl.ds(16 * k, 16)
                rows[e16 * 16 + j, sl] = rows[e16 * 16 + j, sl] * wj

    eb0 = 2 * wid * CE
    pltpu.sync_copy(src_hbm.at[pl.ds(eb0, CE2)], sbuf)
    pltpu.sync_copy(dst_hbm.at[pl.ds(eb0, CE2)], dbuf)
    pltpu.sync_copy(ew_hbm.at[pl.ds(eb0, CE2)], ewbuf)
    pltpu.async_copy(x_hbm.at[sbuf.at[pl.ds(0, CE)]], rows0, sem_g0)
    pltpu.async_copy(rs_sp.at[sbuf.at[pl.ds(0, CE)]], rsga, sem_r0)

    def body(j, carry):
        ebn = 2 * (wid + (j + 1) * NW) * CE
        pltpu.async_copy(x_hbm.at[sbuf.at[pl.ds(CE, CE)]], rows1, sem_g1)
        pltpu.async_copy(rs_sp.at[sbuf.at[pl.ds(CE, CE)]], rsgb, sem_r1)
        pltpu.make_async_copy(x_hbm.at[sbuf.at[pl.ds(0, CE)]], rows0,
                              sem_g0).wait()
        pltpu.make_async_copy(rs_sp.at[sbuf.at[pl.ds(0, CE)]], rsga,
                              sem_r0).wait()
        for k in range(CE // 16):
            sl = pl.ds(16 * k, 16)
            ewbuf[sl] = ewbuf[sl] * rsga[sl]

        def esc_a(e16, c2):
            escale(rows0, ewbuf, e16, 0)
            return c2

        lax.fori_loop(0, CE // 16, esc_a, 0)
        pltpu.sync_copy(rows0, acc.at[dbuf.at[pl.ds(0, CE)]], add=True)
        pltpu.make_async_copy(x_hbm.at[sbuf.at[pl.ds(CE, CE)]], rows1,
                              sem_g1).wait()
        pltpu.make_async_copy(rs_sp.at[sbuf.at[pl.ds(CE, CE)]], rsgb,
                              sem_r1).wait()
        for k in range(CE // 16):
            sl = pl.ds(16 * k, 16)
            slb = pl.ds(CE + 16 * k, 16)
            db2[sl] = dbuf[slb]
            ewb2[sl] = ewbuf[slb] * rsgb[sl]

        @pl.when(j + 1 < npairs)
        def _next():
            pltpu.sync_copy(src_hbm.at[pl.ds(ebn, CE2)], sbuf)
            pltpu.sync_copy(dst_hbm.at[pl.ds(ebn, CE2)], dbuf)
            pltpu.sync_copy(ew_hbm.at[pl.ds(ebn, CE2)], ewbuf)
            pltpu.async_copy(x_hbm.at[sbuf.at[pl.ds(0, CE)]], rows0, sem_g0)
            pltpu.async_copy(rs_sp.at[sbuf.at[pl.ds(0, CE)]], rsga, sem_r0)

        def esc_b(e16, c2):
            escale(rows1, ewb2, e16, 0)
            return c2

        lax.fori_loop(0, CE // 16, esc_b, 0)
        pltpu.sync_copy(rows1, acc.at[db2], add=True)
        return carry

    lax.fori_loop(0, npairs, body, 0)
    plsc.subcore_barrier()
    for j in range(ZB // 64):
        pltpu.sync_copy(acc.at[pl.ds(sid * ZB + j * 64, 64), :],
                        out_hbm.at[cid, pl.ds(sid * ZB + j * 64, 64), :])


def _leaky(z):
    return jnp.where(z >= 0, z, 0.01 * z)


_HI = lax.Precision.HIGHEST


BN = 2000


def _mom_body(p_ref, degp_ref, nw_ref, gid_ref, wc_ref,
              gg_ref, gb_ref, ga_ref, wl_ref, wcls_ref,
              s1_ref, s2_ref, t1_ref, c_ref, out_ref):
    f32 = jnp.float32
    agg = p_ref[0] + p_ref[1]
    din = degp_ref[1] + degp_ref[3]
    sin = lax.rsqrt(jnp.maximum(din, 1.0))
    h = jnp.dot(agg, wc_ref[...], preferred_element_type=f32,
                precision=_HI) * sin
    h = _leaky(h)

    ids = gid_ref[...]
    iota_g = lax.broadcasted_iota(jnp.int32, (BN, G), 1)
    m = (ids == iota_g).astype(f32)

    seg = lambda v: lax.dot_general(
        m, v, (((0,), (0,)), ((), ())), preferred_element_type=f32,
        precision=_HI)

    nw = nw_ref[...]
    onw = jnp.concatenate([jnp.ones((BN, 1), f32), nw], axis=1)

    @pl.when(pl.program_id(0) == 0)
    def _init():
        s1_ref[...] = jnp.zeros_like(s1_ref)
        s2_ref[...] = jnp.zeros_like(s2_ref)
        t1_ref[...] = jnp.zeros_like(t1_ref)
        c_ref[...] = jnp.zeros_like(c_ref)

    s1_ref[...] += seg(h)
    s2_ref[...] += seg(h * h)
    t1_ref[...] += seg(nw * h)
    c_ref[...] += seg(onw)

    @pl.when(pl.program_id(0) == N // BN - 1)
    def _fin():
        cnt = jnp.maximum(c_ref[:, 0:1], 1.0)
        wg = c_ref[:, 1:2]
        inv = 1.0 / cnt
        a = ga_ref[...]
        mean = s1_ref[...] * inv
        var = s2_ref[...] * inv - (2.0 * a - a * a) * mean * mean
        hscale = gg_ref[...] * lax.rsqrt(var + 1e-5)
        r = (hscale * (t1_ref[...] - a * mean * wg) + gb_ref[...] * wg) * inv
        r2 = _leaky(jnp.dot(r, wl_ref[...], preferred_element_type=f32,
                            precision=_HI))
        mu = jnp.mean(r2, axis=1, keepdims=True)
        v = jnp.mean((r2 - mu) ** 2, axis=1, keepdims=True)
        rn = (r2 - mu) * lax.rsqrt(v + 1e-5)
        out_ref[...] = jnp.dot(rn, wcls_ref[...], preferred_element_type=f32,
                               precision=_HI)


@functools.lru_cache(maxsize=None)
def _build_sc_kernels():
    mesh = plsc.VectorSubcoreMesh(
        core_axis_name="c", subcore_axis_name="s",
        num_cores=NC, num_subcores=NS)
    deg = pl.kernel(
        _deg_body,
        out_type=jax.ShapeDtypeStruct((NC, 2, NP), jnp.float32),
        mesh=mesh,
        scratch_types=[
            pltpu.VMEM((CE2,), jnp.int32),
            pltpu.VMEM((CE2,), jnp.int32),
            pltpu.VMEM((CE2,), jnp.int32),
            pltpu.VMEM((CE2,), jnp.int32),
            pltpu.VMEM((CE,), jnp.float32),
            pltpu.VMEM((ZB,), jnp.float32),
            pltpu.VMEM_SHARED((NP,), jnp.float32),
            pltpu.VMEM_SHARED((NP,), jnp.float32),
            pltpu.SemaphoreType.DMA,
            pltpu.SemaphoreType.DMA,
            pltpu.SemaphoreType.DMA,
            pltpu.SemaphoreType.DMA,
        ],
    )
    spmm = pl.kernel(
        _spmm_body,
        out_type=jax.ShapeDtypeStruct((NC, NP, F), jnp.float32),
        mesh=mesh,
        scratch_types=[
            pltpu.VMEM((CE2,), jnp.int32),
            pltpu.VMEM((CE2,), jnp.int32),
            pltpu.VMEM((CE2,), jnp.float32),
            pltpu.VMEM((CE,), jnp.int32),
            pltpu.VMEM((CE,), jnp.float32),
            pltpu.VMEM((CE,), jnp.float32),
            pltpu.VMEM((CE,), jnp.float32),
            pltpu.VMEM((ZB,), jnp.float32),
            pltpu.VMEM((ZB,), jnp.float32),
            pltpu.VMEM((ZB,), jnp.float32),
            pltpu.VMEM((CE, F), jnp.float32),
            pltpu.VMEM((CE, F), jnp.float32),
            pltpu.VMEM_SHARED((NP, F), jnp.float32),
            pltpu.VMEM_SHARED((NP,), jnp.float32),
            pltpu.SemaphoreType.DMA,
            pltpu.SemaphoreType.DMA,
            pltpu.SemaphoreType.DMA,
            pltpu.SemaphoreType.DMA,
        ],
    )
    return deg, spmm


def kernel(x, edge_index, edge_weight, node_weight, graph_ids, W_conv,
           gn_gamma, gn_beta, gn_alpha, W_lin, W_cls):
    src = edge_index[0]
    dst = edge_index[1]
    _deg_kernel, _spmm_kernel = _build_sc_kernels()

    degp = _deg_kernel(src, dst)
    degp4 = degp.reshape(4, NP, 1)

    p = _spmm_kernel(src, dst, edge_weight, x, degp)

    outs = pl.pallas_call(
        _mom_body,
        grid=(N // BN,),
        in_specs=[
            pl.BlockSpec((2, BN, F), lambda i: (0, i, 0)),
            pl.BlockSpec((4, BN, 1), lambda i: (0, i, 0)),
            pl.BlockSpec((BN, 1), lambda i: (i, 0)),
            pl.BlockSpec((BN, 1), lambda i: (i, 0)),
            pl.BlockSpec((F, H), lambda i: (0, 0)),
            pl.BlockSpec((1, H), lambda i: (0, 0)),
            pl.BlockSpec((1, H), lambda i: (0, 0)),
            pl.BlockSpec((1, H), lambda i: (0, 0)),
            pl.BlockSpec((H, H // 2), lambda i: (0, 0)),
            pl.BlockSpec((H // 2, OUT), lambda i: (0, 0)),
        ],
        out_specs=[
            pl.BlockSpec((G, H), lambda i: (0, 0)),
            pl.BlockSpec((G, H), lambda i: (0, 0)),
            pl.BlockSpec((G, H), lambda i: (0, 0)),
            pl.BlockSpec((G, 2), lambda i: (0, 0)),
            pl.BlockSpec((G, OUT), lambda i: (0, 0)),
        ],
        out_shape=[
            jax.ShapeDtypeStruct((G, H), jnp.float32),
            jax.ShapeDtypeStruct((G, H), jnp.float32),
            jax.ShapeDtypeStruct((G, H), jnp.float32),
            jax.ShapeDtypeStruct((G, 2), jnp.float32),
            jax.ShapeDtypeStruct((G, OUT), jnp.float32),
        ],
    )(p, degp4, node_weight.reshape(N, 1), graph_ids.reshape(N, 1), W_conv,
      gn_gamma.reshape(1, H), gn_beta.reshape(1, H), gn_alpha.reshape(1, H),
      W_lin, W_cls)
    return outs[4]


# speedup vs baseline: 13.7408x; 1.1473x over previous
import functools

import jax
import jax.numpy as jnp
from jax import lax
from jax.experimental import pallas as pl
from jax.experimental.pallas import tpu as pltpu
from jax.experimental.pallas import tpu_sc as plsc

N = 10000
E = 320000
F = 128
H = 256
G = 64
OUT = 10

NC = 2
NS = 16
NW = NC * NS
CE = 128
NCHUNK = E // CE
NP = 10240
ZB = NP // NS

NPAIR = NCHUNK // 2
PB = NPAIR // NW
PR = NPAIR % NW
NQ = NCHUNK // 4
QB = NQ // NW
QR = NQ % NW

CE2 = 2 * CE


def _deg_body(src_hbm, dst_hbm, out_hbm, sa, da, sb, db, ones, zbuf,
              acc_src, acc_dst, sem_sa, sem_da, sem_sb, sem_db):
    cid = lax.axis_index("c")
    sid = lax.axis_index("s")
    wid = sid * NC + cid
    for i in range(CE // 16):
        ones[pl.ds(16 * i, 16)] = jnp.ones((16,), jnp.float32)
    for i in range(ZB // 16):
        zbuf[pl.ds(16 * i, 16)] = jnp.zeros((16,), jnp.float32)
    pltpu.sync_copy(zbuf, acc_src.at[pl.ds(sid * ZB, ZB)])
    pltpu.sync_copy(zbuf, acc_dst.at[pl.ds(sid * ZB, ZB)])
    plsc.subcore_barrier()

    nq = QB + jnp.where(wid < QR, 1, 0)

    eb0 = 4 * wid * CE
    pltpu.async_copy(src_hbm.at[pl.ds(eb0, CE2)], sa, sem_sa)
    pltpu.async_copy(dst_hbm.at[pl.ds(eb0, CE2)], da, sem_da)

    def body(q, carry):
        eb = 4 * (wid + q * NW) * CE
        ebn = 4 * (wid + (q + 1) * NW) * CE
        pltpu.make_async_copy(src_hbm.at[pl.ds(eb, CE2)], sa, sem_sa).wait()
        pltpu.make_async_copy(dst_hbm.at[pl.ds(eb, CE2)], da, sem_da).wait()
        pltpu.async_copy(src_hbm.at[pl.ds(eb + CE2, CE2)], sb, sem_sb)
        pltpu.async_copy(dst_hbm.at[pl.ds(eb + CE2, CE2)], db, sem_db)
        pltpu.sync_copy(ones, acc_src.at[sa.at[pl.ds(0, CE)]], add=True)
        pltpu.sync_copy(ones, acc_src.at[sa.at[pl.ds(CE, CE)]], add=True)
        pltpu.sync_copy(ones, acc_dst.at[da.at[pl.ds(0, CE)]], add=True)
        pltpu.sync_copy(ones, acc_dst.at[da.at[pl.ds(CE, CE)]], add=True)
        pltpu.make_async_copy(src_hbm.at[pl.ds(eb + CE2, CE2)], sb,
                              sem_sb).wait()
        pltpu.make_async_copy(dst_hbm.at[pl.ds(eb + CE2, CE2)], db,
                              sem_db).wait()

        @pl.when(q + 1 < nq)
        def _next():
            pltpu.async_copy(src_hbm.at[pl.ds(ebn, CE2)], sa, sem_sa)
            pltpu.async_copy(dst_hbm.at[pl.ds(ebn, CE2)], da, sem_da)

        pltpu.sync_copy(ones, acc_src.at[sb.at[pl.ds(0, CE)]], add=True)
        pltpu.sync_copy(ones, acc_src.at[sb.at[pl.ds(CE, CE)]], add=True)
        pltpu.sync_copy(ones, acc_dst.at[db.at[pl.ds(0, CE)]], add=True)
        pltpu.sync_copy(ones, acc_dst.at[db.at[pl.ds(CE, CE)]], add=True)
        return carry

    lax.fori_loop(0, nq, body, 0)
    plsc.subcore_barrier()
    pltpu.sync_copy(acc_src.at[pl.ds(sid * ZB, ZB)],
                    out_hbm.at[cid, 0, pl.ds(sid * ZB, ZB)])
    pltpu.sync_copy(acc_dst.at[pl.ds(sid * ZB, ZB)],
                    out_hbm.at[cid, 1, pl.ds(sid * ZB, ZB)])


_RSQRT_MAGIC = 0x5F3759DF


def _newton_rsqrt(m):
    bi = lax.bitcast_convert_type(m, jnp.int32)
    y = lax.bitcast_convert_type(
        jnp.full((16,), _RSQRT_MAGIC, jnp.int32)
        - lax.shift_right_arithmetic(bi, jnp.full((16,), 1, jnp.int32)),
        jnp.float32)
    half = m * (-0.5)
    for _ in range(3):
        y = y * (half * y * y + 1.5)
    return y


def _spmm_body(src_hbm, dst_hbm, ew_hbm, x_hbm, degp_hbm, out_hbm,
               sbuf, dbuf, ewbuf, db2, ewb2, rsga, rsgb, hb0, hb1, rsl,
               rows0, rows1, acc, rs_sp,
               sem_g0, sem_g1, sem_r0, sem_r1, sem_ls, sem_ld, sem_lw,
               sem_sa):
    cid = lax.axis_index("c")
    sid = lax.axis_index("s")
    wid = sid * NC + cid

    pltpu.sync_copy(degp_hbm.at[0, 0, pl.ds(sid * ZB, ZB)], hb0)
    pltpu.sync_copy(degp_hbm.at[1, 0, pl.ds(sid * ZB, ZB)], hb1)

    def rsloop(i, carry):
        sl = pl.ds(i * 16, 16)
        m = jnp.maximum(hb0[sl] + hb1[sl], 1.0)
        rsl[sl] = _newton_rsqrt(m)
        return carry

    lax.fori_loop(0, ZB // 16, rsloop, 0)
    pltpu.sync_copy(rsl, rs_sp.at[pl.ds(sid * ZB, ZB)])

    def zfill(i, carry):
        for k in range(F // 16):
            rows0[i, pl.ds(16 * k, 16)] = jnp.zeros((16,), jnp.float32)
        return carry

    lax.fori_loop(0, CE, zfill, 0)
    for j in range(ZB // CE):
        pltpu.sync_copy(rows0, acc.at[pl.ds(sid * ZB + j * CE, CE), :])
    plsc.subcore_barrier()

    npairs = PB + jnp.where(wid < PR, 1, 0)

    def escale(rows, ewb, e16, lo):
        wv = ewb[pl.ds(lo + e16 * 16, 16)]
        for j in range(16):
            wj = wv.at[jnp.full((16,), j, jnp.int32)].get(
                mode='promise_in_bounds')
            for k in range(F // 16):
                sl = pl.ds(16 * k, 16)
                rows[e16 * 16 + j, sl] = rows[e16 * 16 + j, sl] * wj

    eb0 = 2 * wid * CE
    pltpu.async_copy(src_hbm.at[pl.ds(eb0, CE2)], sbuf, sem_ls)
    pltpu.async_copy(dst_hbm.at[pl.ds(eb0, CE2)], dbuf, sem_ld)
    pltpu.async_copy(ew_hbm.at[pl.ds(eb0, CE2)], ewbuf, sem_lw)
    pltpu.make_async_copy(src_hbm.at[pl.ds(eb0, CE2)], sbuf, sem_ls).wait()
    pltpu.async_copy(x_hbm.at[sbuf.at[pl.ds(0, CE)]], rows0, sem_g0)
    pltpu.async_copy(rs_sp.at[sbuf.at[pl.ds(0, CE)]], rsga, sem_r0)

    def body(j, carry):
        eb = 2 * (wid + j * NW) * CE
        ebn = 2 * (wid + (j + 1) * NW) * CE
        pltpu.make_async_copy(dst_hbm.at[pl.ds(eb, CE2)], dbuf,
                              sem_ld).wait()
        pltpu.make_async_copy(ew_hbm.at[pl.ds(eb, CE2)], ewbuf,
                              sem_lw).wait()
        pltpu.async_copy(x_hbm.at[sbuf.at[pl.ds(CE, CE)]], rows1, sem_g1)
        pltpu.async_copy(rs_sp.at[sbuf.at[pl.ds(CE, CE)]], rsgb, sem_r1)
        pltpu.make_async_copy(x_hbm.at[sbuf.at[pl.ds(0, CE)]], rows0,
                              sem_g0).wait()
        pltpu.make_async_copy(rs_sp.at[sbuf.at[pl.ds(0, CE)]], rsga,
                              sem_r0).wait()
        for k in range(CE // 16):
            sl = pl.ds(16 * k, 16)
            ewbuf[sl] = ewbuf[sl] * rsga[sl]

        def esc_a(e16, c2):
            escale(rows0, ewbuf, e16, 0)
            return c2

        lax.fori_loop(0, CE // 16, esc_a, 0)
        pltpu.async_copy(rows0, acc.at[dbuf.at[pl.ds(0, CE)]], sem_sa,
                         add=True)
        pltpu.make_async_copy(x_hbm.at[sbuf.at[pl.ds(CE, CE)]], rows1,
                              sem_g1).wait()
        pltpu.make_async_copy(rs_sp.at[sbuf.at[pl.ds(CE, CE)]], rsgb,
                              sem_r1).wait()
        for k in range(CE // 16):
            sl = pl.ds(16 * k, 16)
            slb = pl.ds(CE + 16 * k, 16)
            db2[sl] = dbuf[slb]
            ewb2[sl] = ewbuf[slb] * rsgb[sl]
        pltpu.make_async_copy(rows0, acc.at[dbuf.at[pl.ds(0, CE)]],
                              sem_sa).wait()

        @pl.when(j + 1 < npairs)
        def _next():
            pltpu.async_copy(src_hbm.at[pl.ds(ebn, CE2)], sbuf, sem_ls)
            pltpu.async_copy(dst_hbm.at[pl.ds(ebn, CE2)], dbuf, sem_ld)
            pltpu.async_copy(ew_hbm.at[pl.ds(ebn, CE2)], ewbuf, sem_lw)
            pltpu.make_async_copy(src_hbm.at[pl.ds(ebn, CE2)], sbuf,
                                  sem_ls).wait()
            pltpu.async_copy(x_hbm.at[sbuf.at[pl.ds(0, CE)]], rows0, sem_g0)
            pltpu.async_copy(rs_sp.at[sbuf.at[pl.ds(0, CE)]], rsga, sem_r0)

        def esc_b(e16, c2):
            escale(rows1, ewb2, e16, 0)
            return c2

        lax.fori_loop(0, CE // 16, esc_b, 0)
        pltpu.sync_copy(rows1, acc.at[db2], add=True)
        return carry

    lax.fori_loop(0, npairs, body, 0)
    plsc.subcore_barrier()
    for j in range(ZB // 64):
        pltpu.sync_copy(acc.at[pl.ds(sid * ZB + j * 64, 64), :],
                        out_hbm.at[cid, pl.ds(sid * ZB + j * 64, 64), :])


def _leaky(z):
    return jnp.where(z >= 0, z, 0.01 * z)


_HI = lax.Precision.HIGHEST


BN = 2000


def _mom_body(p_ref, degp_ref, nw_ref, gid_ref, wc_ref,
              gg_ref, gb_ref, ga_ref, wl_ref, wcls_ref,
              s1_ref, s2_ref, t1_ref, c_ref, out_ref):
    f32 = jnp.float32
    agg = p_ref[0] + p_ref[1]
    din = degp_ref[1] + degp_ref[3]
    sin = lax.rsqrt(jnp.maximum(din, 1.0))
    h = jnp.dot(agg, wc_ref[...], preferred_element_type=f32,
                precision=_HI) * sin
    h = _leaky(h)

    ids = gid_ref[...]
    iota_g = lax.broadcasted_iota(jnp.int32, (BN, G), 1)
    m = (ids == iota_g).astype(f32)

    seg = lambda v: lax.dot_general(
        m, v, (((0,), (0,)), ((), ())), preferred_element_type=f32,
        precision=_HI)

    nw = nw_ref[...]
    onw = jnp.concatenate([jnp.ones((BN, 1), f32), nw], axis=1)

    @pl.when(pl.program_id(0) == 0)
    def _init():
        s1_ref[...] = jnp.zeros_like(s1_ref)
        s2_ref[...] = jnp.zeros_like(s2_ref)
        t1_ref[...] = jnp.zeros_like(t1_ref)
        c_ref[...] = jnp.zeros_like(c_ref)

    s1_ref[...] += seg(h)
    s2_ref[...] += seg(h * h)
    t1_ref[...] += seg(nw * h)
    c_ref[...] += seg(onw)

    @pl.when(pl.program_id(0) == N // BN - 1)
    def _fin():
        cnt = jnp.maximum(c_ref[:, 0:1], 1.0)
        wg = c_ref[:, 1:2]
        inv = 1.0 / cnt
        a = ga_ref[...]
        mean = s1_ref[...] * inv
        var = s2_ref[...] * inv - (2.0 * a - a * a) * mean * mean
        hscale = gg_ref[...] * lax.rsqrt(var + 1e-5)
        r = (hscale * (t1_ref[...] - a * mean * wg) + gb_ref[...] * wg) * inv
        r2 = _leaky(jnp.dot(r, wl_ref[...], preferred_element_type=f32,
                            precision=_HI))
        mu = jnp.mean(r2, axis=1, keepdims=True)
        v = jnp.mean((r2 - mu) ** 2, axis=1, keepdims=True)
        rn = (r2 - mu) * lax.rsqrt(v + 1e-5)
        out_ref[...] = jnp.dot(rn, wcls_ref[...], preferred_element_type=f32,
                               precision=_HI)


@functools.lru_cache(maxsize=None)
def _build_sc_kernels():
    mesh = plsc.VectorSubcoreMesh(
        core_axis_name="c", subcore_axis_name="s",
        num_cores=NC, num_subcores=NS)
    deg = pl.kernel(
        _deg_body,
        out_type=jax.ShapeDtypeStruct((NC, 2, NP), jnp.float32),
        mesh=mesh,
        scratch_types=[
            pltpu.VMEM((CE2,), jnp.int32),
            pltpu.VMEM((CE2,), jnp.int32),
            pltpu.VMEM((CE2,), jnp.int32),
            pltpu.VMEM((CE2,), jnp.int32),
            pltpu.VMEM((CE,), jnp.float32),
            pltpu.VMEM((ZB,), jnp.float32),
            pltpu.VMEM_SHARED((NP,), jnp.float32),
            pltpu.VMEM_SHARED((NP,), jnp.float32),
            pltpu.SemaphoreType.DMA,
            pltpu.SemaphoreType.DMA,
            pltpu.SemaphoreType.DMA,
            pltpu.SemaphoreType.DMA,
        ],
    )
    spmm = pl.kernel(
        _spmm_body,
        out_type=jax.ShapeDtypeStruct((NC, NP, F), jnp.float32),
        mesh=mesh,
        scratch_types=[
            pltpu.VMEM((CE2,), jnp.int32),
            pltpu.VMEM((CE2,), jnp.int32),
            pltpu.VMEM((CE2,), jnp.float32),
            pltpu.VMEM((CE,), jnp.int32),
            pltpu.VMEM((CE,), jnp.float32),
            pltpu.VMEM((CE,), jnp.float32),
            pltpu.VMEM((CE,), jnp.float32),
            pltpu.VMEM((ZB,), jnp.float32),
            pltpu.VMEM((ZB,), jnp.float32),
            pltpu.VMEM((ZB,), jnp.float32),
            pltpu.VMEM((CE, F), jnp.float32),
            pltpu.VMEM((CE, F), jnp.float32),
            pltpu.VMEM_SHARED((NP, F), jnp.float32),
            pltpu.VMEM_SHARED((NP,), jnp.float32),
            pltpu.SemaphoreType.DMA,
            pltpu.SemaphoreType.DMA,
            pltpu.SemaphoreType.DMA,
            pltpu.SemaphoreType.DMA,
            pltpu.SemaphoreType.DMA,
            pltpu.SemaphoreType.DMA,
            pltpu.SemaphoreType.DMA,
            pltpu.SemaphoreType.DMA,
        ],
    )
    return deg, spmm


def kernel(x, edge_index, edge_weight, node_weight, graph_ids, W_conv,
           gn_gamma, gn_beta, gn_alpha, W_lin, W_cls):
    src = edge_index[0]
    dst = edge_index[1]
    _deg_kernel, _spmm_kernel = _build_sc_kernels()

    degp = _deg_kernel(src, dst)
    degp4 = degp.reshape(4, NP, 1)

    p = _spmm_kernel(src, dst, edge_weight, x, degp)

    outs = pl.pallas_call(
        _mom_body,
        grid=(N // BN,),
        in_specs=[
            pl.BlockSpec((2, BN, F), lambda i: (0, i, 0)),
            pl.BlockSpec((4, BN, 1), lambda i: (0, i, 0)),
            pl.BlockSpec((BN, 1), lambda i: (i, 0)),
            pl.BlockSpec((BN, 1), lambda i: (i, 0)),
            pl.BlockSpec((F, H), lambda i: (0, 0)),
            pl.BlockSpec((1, H), lambda i: (0, 0)),
            pl.BlockSpec((1, H), lambda i: (0, 0)),
            pl.BlockSpec((1, H), lambda i: (0, 0)),
            pl.BlockSpec((H, H // 2), lambda i: (0, 0)),
            pl.BlockSpec((H // 2, OUT), lambda i: (0, 0)),
        ],
        out_specs=[
            pl.BlockSpec((G, H), lambda i: (0, 0)),
            pl.BlockSpec((G, H), lambda i: (0, 0)),
            pl.BlockSpec((G, H), lambda i: (0, 0)),
            pl.BlockSpec((G, 2), lambda i: (0, 0)),
            pl.BlockSpec((G, OUT), lambda i: (0, 0)),
        ],
        out_shape=[
            jax.ShapeDtypeStruct((G, H), jnp.float32),
            jax.ShapeDtypeStruct((G, H), jnp.float32),
            jax.ShapeDtypeStruct((G, H), jnp.float32),
            jax.ShapeDtypeStruct((G, 2), jnp.float32),
            jax.ShapeDtypeStruct((G, OUT), jnp.float32),
        ],
    )(p, degp4, node_weight.reshape(N, 1), graph_ids.reshape(N, 1), W_conv,
      gn_gamma.reshape(1, H), gn_beta.reshape(1, H), gn_alpha.reshape(1, H),
      W_lin, W_cls)
    return outs[4]
